# Initial kernel scaffold; baseline (speedup 1.0000x reference)
import jax
import jax.numpy as jnp
from jax.experimental import pallas as pl


def kernel(inputs, mask, Di_vals, DiA_vals, params, Di_rows, Di_cols, DiA_rows, DiA_cols):
    raise NotImplementedError("write your pallas kernel here")


# speedup vs baseline: 1.0006x; 1.0006x over previous
import jax
import jax.numpy as jnp
from jax.experimental import pallas as pl
from jax.experimental.pallas import tpu as pltpu

B = 1
N = 10000
NF = 20000
C = 128
NNZ = 960000


def _elu(x):
    return jnp.where(x > 0, x, jnp.exp(jnp.minimum(x, 0.0)) - 1.0)


def _graph_conv1x1(x, p, bn):
    b, n, c = x.shape
    if bn:
        xf = x.reshape(-1, c)
        mu = jnp.mean(xf, 0)
        var = jnp.var(xf, 0)
        xf = (xf - mu) * jax.lax.rsqrt(var + 1e-5)
        xf = xf * p["gamma"] + p["beta"]
        x = xf.reshape(b, n, c)
    return x @ p["W"] + p["b"]


def _sparse_bmm(rows, cols, vals, dense, nrows):
    def one(d):
        g = jnp.take(d, cols, axis=0) * vals[:, None]
        return jax.ops.segment_sum(g, rows, num_segments=nrows)
    return jax.vmap(one)(dense)


def _masked_avg(x, mask):
    m = (x * mask).sum(1, keepdims=True) / mask.sum(1, keepdims=True)
    return jnp.broadcast_to(m, x.shape)


def _final_conv_body(x_ref, w_ref, b_ref, g_ref, bt_ref, o_ref):
    x = _elu(x_ref[...])
    mu = jnp.mean(x, axis=0, keepdims=True)
    var = jnp.mean(jnp.square(x - mu), axis=0, keepdims=True)
    xn = (x - mu) * jax.lax.rsqrt(var + 1e-5)
    xn = xn * g_ref[...] + bt_ref[...]
    o = xn @ w_ref[...] + b_ref[0, 0]
    o_ref[...] = jnp.broadcast_to(o, (NF, 128))


def _final_conv(f, p):
    x = f.reshape(NF, C)
    w = p["W"].reshape(C, 1)
    out = pl.pallas_call(
        _final_conv_body,
        out_shape=jax.ShapeDtypeStruct((NF, 128), jnp.float32),
    )(x, w, p["b"].reshape(1, 1), p["gamma"].reshape(1, C), p["beta"].reshape(1, C))
    return out[:, :1].reshape(B, NF, 1)


def kernel(inputs, mask, Di_vals, DiA_vals, params, Di_rows, Di_cols, DiA_rows, DiA_cols):
    v = _graph_conv1x1(inputs, params["conv1"], False)
    f = jnp.zeros((B, NF, C), jnp.float32)
    for i in range(16):
        blk = params["blocks"][i]
        if i % 2 == 0:
            x_in = v
            x = _elu(v)
            fe = _elu(f)
            x4 = x.reshape(B, 4 * N, C // 4)
            f4 = fe.reshape(B, 4 * NF, C // 4)
            Dv = _sparse_bmm(Di_rows, Di_cols, Di_vals, x4, 4 * NF).reshape(B, NF, C)
            DAf = _sparse_bmm(DiA_rows, DiA_cols, DiA_vals, f4, 4 * N).reshape(B, N, C)
            v = _graph_conv1x1(jnp.concatenate([x, DAf], 2), blk["fc0"], True) + x_in
            f = _graph_conv1x1(jnp.concatenate([fe, Dv], 2), blk["fc1"], True)
        else:
            x_in = v
            x = _elu(v)
            x = jnp.concatenate([x, _masked_avg(x, mask)], 2)
            x = _graph_conv1x1(x, blk["fc0"], True)
            x = _elu(x)
            x = jnp.concatenate([x, _masked_avg(x, mask)], 2)
            x = _graph_conv1x1(x, blk["fc1"], True)
            v = x + x_in
    return _final_conv(f, params["conv2"])


# speedup vs baseline: 2.9726x; 2.9709x over previous
import functools

import jax
import jax.numpy as jnp
from jax import lax
from jax.experimental import pallas as pl
from jax.experimental.pallas import tpu as pltpu
from jax.experimental.pallas import tpu_sc as plsc
from jax._src import core as _jax_core
from jax._src.pallas import core as _pallas_core


def _to_device_space(x):
    return _pallas_core.with_memory_space_constraint_p.bind(
        x, memory_space=_jax_core.MemorySpace.Device)

B = 1
N = 10000
NF = 20000
C = 128
NNZ = 960000

_CHUNK = 128
_NNZ_P = 983040
_NCHUNK = _NNZ_P // _CHUNK
_PER_SUB = _NCHUNK // 16
_SUP = 24
_NSUP = _PER_SUB // _SUP


def _pad_edges(a):
    return jnp.concatenate([a, jnp.zeros((_NNZ_P - NNZ,), a.dtype)])


def _make_sc_spmm(M, R, npass):
    RH = R // npass
    stripe = -(-(RH // 16 + 1) // 8) * 8
    RHp = 16 * stripe

    grid = plsc.VectorSubcoreMesh(core_axis_name="c", subcore_axis_name="s")

    @functools.partial(
        pl.kernel,
        out_type=pltpu.HBM((2, npass, RHp, 16), jnp.float32),
        mesh=grid,
        compiler_params=pltpu.CompilerParams(use_tc_tiling_on_sc=False),
        scratch_types=[
            pltpu.VMEM_SHARED((RHp, 16), jnp.float32),
            pltpu.VMEM((stripe, 16), jnp.float32),
            pltpu.VMEM((_SUP, _CHUNK), jnp.int32),
            pltpu.VMEM((_SUP, _CHUNK), jnp.float32),
            pltpu.VMEM((_SUP, _CHUNK), jnp.int32),
            pltpu.VMEM((_CHUNK, 16), jnp.float32),
            pltpu.SemaphoreType.DMA,
        ],
    )
    def spmm(tbl, cols, vals, rows, out, acc, zbuf, cidx, vbuf, ridx, g, gsem):
        c = lax.axis_index("c")
        s = lax.axis_index("s")
        z16 = jnp.zeros((16,), jnp.float32)

        def zrow(r, carry):
            zbuf[r, pl.ds(0, 16)] = z16
            return carry

        lax.fori_loop(0, stripe, zrow, 0)

        cM = (c * M).astype(jnp.int32)
        base_chunk = s * _PER_SUB

        for p in range(npass):
            lo = jnp.int32(p * RH)
            pltpu.sync_copy(zbuf, acc.at[pl.ds(s * stripe, stripe)])
            plsc.subcore_barrier()

            def super_body(t, carry):
                row0 = base_chunk + t * _SUP
                pltpu.sync_copy(cols.at[pl.ds(row0, _SUP)], cidx)
                pltpu.sync_copy(vals.at[pl.ds(row0, _SUP)], vbuf)
                pltpu.sync_copy(rows.at[pl.ds(row0, _SUP)], ridx)

                def chunk_body(q, carry2):
                    for k in range(_CHUNK // 16):
                        sl = pl.ds(k * 16, 16)
                        cidx[q, sl] = cidx[q, sl] + cM
                        if npass > 1:
                            lr = ridx[q, sl] - lo
                            ok = (lr >= 0) & (lr < RH)
                            ridx[q, sl] = jnp.where(ok, lr, jnp.int32(RH))
                    pltpu.async_copy(tbl.at[cidx.at[q]], g, gsem).wait()
                    for k in range(_CHUNK // 16):
                        val16 = vbuf[q, pl.ds(k * 16, 16)]
                        for j in range(16):
                            e = k * 16 + j
                            g[e, pl.ds(0, 16)] = g[e, pl.ds(0, 16)] * val16[j]
                    pltpu.sync_copy(g, acc.at[ridx.at[q]], add=True)
                    return carry2

                lax.fori_loop(0, _SUP, chunk_body, 0)
                return carry

            lax.fori_loop(0, _NSUP, super_body, 0)
            plsc.subcore_barrier()
            pltpu.sync_copy(acc.at[pl.ds(s * stripe, stripe)],
                            out.at[c, p, pl.ds(s * stripe, stripe)])
            if p + 1 < npass:
                plsc.subcore_barrier()

    return spmm


_sc_spmm_di = _make_sc_spmm(4 * N, 4 * NF, 2)
_sc_spmm_dia = _make_sc_spmm(4 * NF, 4 * N, 1)


def _sc_sparse_bmm(spmm, x4, cols2, vals2, rows2, R, npass):
    tbl = jnp.concatenate([x4[0, :, :16], x4[0, :, 16:]], 0)
    out2 = _to_device_space(spmm(tbl, cols2, vals2, rows2))
    RH = R // npass
    halves = [out2[:, p, :RH] for p in range(npass)]
    full = jnp.concatenate(halves, 1)
    return jnp.concatenate([full[0], full[1]], 1)[None]


def _elu(x):
    return jnp.where(x > 0, x, jnp.exp(jnp.minimum(x, 0.0)) - 1.0)


def _graph_conv1x1(x, p, bn):
    b, n, c = x.shape
    if bn:
        xf = x.reshape(-1, c)
        mu = jnp.mean(xf, 0)
        var = jnp.var(xf, 0)
        xf = (xf - mu) * jax.lax.rsqrt(var + 1e-5)
        xf = xf * p["gamma"] + p["beta"]
        x = xf.reshape(b, n, c)
    return x @ p["W"] + p["b"]


def _sparse_bmm(rows, cols, vals, dense, nrows):
    def one(d):
        g = jnp.take(d, cols, axis=0) * vals[:, None]
        return jax.ops.segment_sum(g, rows, num_segments=nrows)
    return jax.vmap(one)(dense)


def _masked_avg(x, mask):
    m = (x * mask).sum(1, keepdims=True) / mask.sum(1, keepdims=True)
    return jnp.broadcast_to(m, x.shape)


def _final_conv_body(x_ref, w_ref, b_ref, g_ref, bt_ref, o_ref):
    x = _elu(x_ref[...])
    mu = jnp.mean(x, axis=0, keepdims=True)
    var = jnp.mean(jnp.square(x - mu), axis=0, keepdims=True)
    xn = (x - mu) * jax.lax.rsqrt(var + 1e-5)
    xn = xn * g_ref[...] + bt_ref[...]
    o = xn @ w_ref[...] + b_ref[0, 0]
    o_ref[...] = jnp.broadcast_to(o, (NF, 128))


def _final_conv(f, p):
    x = f.reshape(NF, C)
    w = p["W"].reshape(C, 1)
    out = pl.pallas_call(
        _final_conv_body,
        out_shape=jax.ShapeDtypeStruct((NF, 128), jnp.float32),
    )(x, w, p["b"].reshape(1, 1), p["gamma"].reshape(1, C), p["beta"].reshape(1, C))
    return out[:, :1].reshape(B, NF, 1)


def kernel(inputs, mask, Di_vals, DiA_vals, params, Di_rows, Di_cols, DiA_rows, DiA_cols):
    di_c = _pad_edges(Di_cols.astype(jnp.int32)).reshape(_NCHUNK, _CHUNK)
    di_v = _pad_edges(Di_vals).reshape(_NCHUNK, _CHUNK)
    di_r = _pad_edges(Di_rows.astype(jnp.int32)).reshape(_NCHUNK, _CHUNK)
    da_c = _pad_edges(DiA_cols.astype(jnp.int32)).reshape(_NCHUNK, _CHUNK)
    da_v = _pad_edges(DiA_vals).reshape(_NCHUNK, _CHUNK)
    da_r = _pad_edges(DiA_rows.astype(jnp.int32)).reshape(_NCHUNK, _CHUNK)
    v = _graph_conv1x1(inputs, params["conv1"], False)
    f = jnp.zeros((B, NF, C), jnp.float32)
    for i in range(16):
        blk = params["blocks"][i]
        if i % 2 == 0:
            x_in = v
            x = _elu(v)
            fe = _elu(f)
            x4 = x.reshape(B, 4 * N, C // 4)
            f4 = fe.reshape(B, 4 * NF, C // 4)
            Dv = _sc_sparse_bmm(_sc_spmm_di, x4, di_c, di_v, di_r, 4 * NF, 2).reshape(B, NF, C)
            DAf = _sc_sparse_bmm(_sc_spmm_dia, f4, da_c, da_v, da_r, 4 * N, 1).reshape(B, N, C)
            v = _graph_conv1x1(jnp.concatenate([x, DAf], 2), blk["fc0"], True) + x_in
            f = _graph_conv1x1(jnp.concatenate([fe, Dv], 2), blk["fc1"], True)
        else:
            x_in = v
            x = _elu(v)
            x = jnp.concatenate([x, _masked_avg(x, mask)], 2)
            x = _graph_conv1x1(x, blk["fc0"], True)
            x = _elu(x)
            x = jnp.concatenate([x, _masked_avg(x, mask)], 2)
            x = _graph_conv1x1(x, blk["fc1"], True)
            v = x + x_in
    return _final_conv(f, params["conv2"])


# speedup vs baseline: 3.4902x; 1.1741x over previous
import functools

import jax
import jax.numpy as jnp
from jax import lax
from jax.experimental import pallas as pl
from jax.experimental.pallas import tpu as pltpu
from jax.experimental.pallas import tpu_sc as plsc
from jax._src import core as _jax_core
from jax._src.pallas import core as _pallas_core


def _to_device_space(x):
    return _pallas_core.with_memory_space_constraint_p.bind(
        x, memory_space=_jax_core.MemorySpace.Device)

B = 1
N = 10000
NF = 20000
C = 128
NNZ = 960000

_CHUNK = 128
_NNZ_P = 983040
_NCHUNK = _NNZ_P // _CHUNK
_PER_SUB = _NCHUNK // 16
_SUP = 24
_NSUP = _PER_SUB // _SUP


def _pad_edges(a):
    return jnp.concatenate([a, jnp.zeros((_NNZ_P - NNZ,), a.dtype)])


def _make_sc_spmm(M, R, npass):
    RH = R // npass
    stripe = -(-(RH // 16 + 1) // 8) * 8
    RHp = 16 * stripe

    grid = plsc.VectorSubcoreMesh(core_axis_name="c", subcore_axis_name="s")

    GB = _CHUNK * 16 * 4
    STB = 3 * _SUP * _CHUNK * 4

    @functools.partial(
        pl.kernel,
        out_type=pltpu.HBM((2, npass, RHp, 16), jnp.float32),
        mesh=grid,
        compiler_params=pltpu.CompilerParams(use_tc_tiling_on_sc=False),
        scratch_types=[
            pltpu.VMEM_SHARED((RHp, 16), jnp.float32),
            pltpu.VMEM((stripe, 16), jnp.float32),
            pltpu.VMEM((_SUP, _CHUNK), jnp.int32),
            pltpu.VMEM((_SUP, _CHUNK), jnp.float32),
            pltpu.VMEM((_SUP, _CHUNK), jnp.int32),
            pltpu.VMEM((_SUP, _CHUNK), jnp.int32),
            pltpu.VMEM((_SUP, _CHUNK), jnp.float32),
            pltpu.VMEM((_SUP, _CHUNK), jnp.int32),
            pltpu.VMEM((_CHUNK, 16), jnp.float32),
            pltpu.VMEM((_CHUNK, 16), jnp.float32),
            pltpu.SemaphoreType.DMA,
            pltpu.SemaphoreType.DMA,
            pltpu.SemaphoreType.DMA,
            pltpu.SemaphoreType.DMA,
            pltpu.SemaphoreType.DMA,
            pltpu.SemaphoreType.DMA,
        ],
    )
    def spmm(tbl, cols, vals, rows, out, acc, zbuf,
             ci0, vb0, ri0, ci1, vb1, ri1, g0, g1,
             gs0, gs1, ss0, ss1, st0, st1):
        c = lax.axis_index("c")
        s = lax.axis_index("s")
        z16 = jnp.zeros((16,), jnp.float32)
        sets = ((ci0, vb0, ri0, st0), (ci1, vb1, ri1, st1))
        gbuf = (g0, g1)
        gsem = (gs0, gs1)
        ssem = (ss0, ss1)

        def zrow(r, carry):
            zbuf[r, pl.ds(0, 16)] = z16
            return carry

        lax.fori_loop(0, stripe, zrow, 0)

        cM = (c * M).astype(jnp.int32)
        base_chunk = s * _PER_SUB

        def stage_start(t, si):
            ci, vb, ri, sem = sets[si]
            row0 = base_chunk + t * _SUP
            pltpu.async_copy(cols.at[pl.ds(row0, _SUP)], ci, sem)
            pltpu.async_copy(vals.at[pl.ds(row0, _SUP)], vb, sem)
            pltpu.async_copy(rows.at[pl.ds(row0, _SUP)], ri, sem)

        def stage_wait(si):
            ci, vb, ri, sem = sets[si]
            for dst, src in ((ci, cols), (vb, vals), (ri, rows)):
                pltpu.make_async_copy(src.at[pl.ds(0, _SUP)], dst, sem).wait()

        def chunk_wait(sem, b):
            pltpu.make_async_copy(tbl.at[pl.ds(0, _CHUNK)], gbuf[b], sem).wait()

        def transform(si, q, lo):
            ci, _, ri, _ = sets[si]
            for k in range(_CHUNK // 16):
                sl = pl.ds(k * 16, 16)
                ci[q, sl] = ci[q, sl] + cM
                if npass > 1:
                    lr = ri[q, sl] - lo
                    ok = (lr >= 0) & (lr < RH)
                    ri[q, sl] = jnp.where(ok, lr, jnp.int32(RH))

        def gather_start(si, q, b):
            pltpu.async_copy(tbl.at[sets[si][0].at[q]], gbuf[b], gsem[b])

        def process(si, q, b):
            _, vb, ri, _ = sets[si]
            g = gbuf[b]
            for k in range(_CHUNK // 16):
                val16 = vb[q, pl.ds(k * 16, 16)]
                for j in range(16):
                    e = k * 16 + j
                    g[e, pl.ds(0, 16)] = g[e, pl.ds(0, 16)] * val16[j]
            pltpu.async_copy(g, acc.at[ri.at[q]], ssem[b], add=True)

        for p in range(npass):
            lo = jnp.int32(p * RH)
            pltpu.sync_copy(zbuf, acc.at[pl.ds(s * stripe, stripe)])
            plsc.subcore_barrier()
            pltpu.sync_copy(cols.at[pl.ds(base_chunk, _SUP)], ci0)
            pltpu.sync_copy(vals.at[pl.ds(base_chunk, _SUP)], vb0)
            pltpu.sync_copy(rows.at[pl.ds(base_chunk, _SUP)], ri0)

            def superchunk(si):
                def run(t):
                    @pl.when(t > 0)
                    def _():
                        chunk_wait(ssem[0], 0)

                    transform(si, 0, lo)
                    gather_start(si, 0, 0)

                    def body2(i2, carry):
                        for qoff, b in ((0, 0), (1, 1)):
                            q = 2 * i2 + qoff
                            o = 1 - b

                            @pl.when((q < _SUP - 1) & ((t > 0) | (q > 0)))
                            def _():
                                chunk_wait(ssem[o], o)

                            @pl.when(q < _SUP - 1)
                            def _():
                                transform(si, q + 1, lo)
                                gather_start(si, q + 1, o)

                            chunk_wait(gsem[b], b)
                            process(si, q, b)
                        return carry

                    lax.fori_loop(0, _SUP // 2, body2, 0)
                return run

            run0 = superchunk(0)
            run1 = superchunk(1)

            def pair(tt, carry):
                t0 = 2 * tt

                @pl.when(tt > 0)
                def _():
                    stage_wait(0)

                stage_start(t0 + 1, 1)
                run0(t0)
                stage_wait(1)

                @pl.when(tt < _NSUP // 2 - 1)
                def _():
                    stage_start(t0 + 2, 0)

                run1(t0 + 1)
                return carry

            lax.fori_loop(0, _NSUP // 2, pair, 0)
            chunk_wait(ss0, 0)
            chunk_wait(ss1, 1)
            plsc.subcore_barrier()
            pltpu.sync_copy(acc.at[pl.ds(s * stripe, stripe)],
                            out.at[c, p, pl.ds(s * stripe, stripe)])
            if p + 1 < npass:
                plsc.subcore_barrier()

    return spmm


_sc_spmm_di = _make_sc_spmm(4 * N, 4 * NF, 2)
_sc_spmm_dia = _make_sc_spmm(4 * NF, 4 * N, 1)


def _sc_sparse_bmm(spmm, x4, cols2, vals2, rows2, R, npass):
    tbl = jnp.concatenate([x4[0, :, :16], x4[0, :, 16:]], 0)
    out2 = _to_device_space(spmm(tbl, cols2, vals2, rows2))
    RH = R // npass
    halves = [out2[:, p, :RH] for p in range(npass)]
    full = jnp.concatenate(halves, 1)
    return jnp.concatenate([full[0], full[1]], 1)[None]


def _elu(x):
    return jnp.where(x > 0, x, jnp.exp(jnp.minimum(x, 0.0)) - 1.0)


def _graph_conv1x1(x, p, bn):
    b, n, c = x.shape
    if bn:
        xf = x.reshape(-1, c)
        mu = jnp.mean(xf, 0)
        var = jnp.var(xf, 0)
        xf = (xf - mu) * jax.lax.rsqrt(var + 1e-5)
        xf = xf * p["gamma"] + p["beta"]
        x = xf.reshape(b, n, c)
    return x @ p["W"] + p["b"]


def _sparse_bmm(rows, cols, vals, dense, nrows):
    def one(d):
        g = jnp.take(d, cols, axis=0) * vals[:, None]
        return jax.ops.segment_sum(g, rows, num_segments=nrows)
    return jax.vmap(one)(dense)


def _masked_avg(x, mask):
    m = (x * mask).sum(1, keepdims=True) / mask.sum(1, keepdims=True)
    return jnp.broadcast_to(m, x.shape)


def _final_conv_body(x_ref, w_ref, b_ref, g_ref, bt_ref, o_ref):
    x = _elu(x_ref[...])
    mu = jnp.mean(x, axis=0, keepdims=True)
    var = jnp.mean(jnp.square(x - mu), axis=0, keepdims=True)
    xn = (x - mu) * jax.lax.rsqrt(var + 1e-5)
    xn = xn * g_ref[...] + bt_ref[...]
    o = xn @ w_ref[...] + b_ref[0, 0]
    o_ref[...] = jnp.broadcast_to(o, (NF, 128))


def _final_conv(f, p):
    x = f.reshape(NF, C)
    w = p["W"].reshape(C, 1)
    out = pl.pallas_call(
        _final_conv_body,
        out_shape=jax.ShapeDtypeStruct((NF, 128), jnp.float32),
    )(x, w, p["b"].reshape(1, 1), p["gamma"].reshape(1, C), p["beta"].reshape(1, C))
    return out[:, :1].reshape(B, NF, 1)


def kernel(inputs, mask, Di_vals, DiA_vals, params, Di_rows, Di_cols, DiA_rows, DiA_cols):
    di_c = _pad_edges(Di_cols.astype(jnp.int32)).reshape(_NCHUNK, _CHUNK)
    di_v = _pad_edges(Di_vals).reshape(_NCHUNK, _CHUNK)
    di_r = _pad_edges(Di_rows.astype(jnp.int32)).reshape(_NCHUNK, _CHUNK)
    da_c = _pad_edges(DiA_cols.astype(jnp.int32)).reshape(_NCHUNK, _CHUNK)
    da_v = _pad_edges(DiA_vals).reshape(_NCHUNK, _CHUNK)
    da_r = _pad_edges(DiA_rows.astype(jnp.int32)).reshape(_NCHUNK, _CHUNK)
    v = _graph_conv1x1(inputs, params["conv1"], False)
    f = jnp.zeros((B, NF, C), jnp.float32)
    for i in range(16):
        blk = params["blocks"][i]
        if i % 2 == 0:
            x_in = v
            x = _elu(v)
            fe = _elu(f)
            x4 = x.reshape(B, 4 * N, C // 4)
            f4 = fe.reshape(B, 4 * NF, C // 4)
            Dv = _sc_sparse_bmm(_sc_spmm_di, x4, di_c, di_v, di_r, 4 * NF, 2).reshape(B, NF, C)
            DAf = _sc_sparse_bmm(_sc_spmm_dia, f4, da_c, da_v, da_r, 4 * N, 1).reshape(B, N, C)
            v = _graph_conv1x1(jnp.concatenate([x, DAf], 2), blk["fc0"], True) + x_in
            f = _graph_conv1x1(jnp.concatenate([fe, Dv], 2), blk["fc1"], True)
        else:
            x_in = v
            x = _elu(v)
            x = jnp.concatenate([x, _masked_avg(x, mask)], 2)
            x = _graph_conv1x1(x, blk["fc0"], True)
            x = _elu(x)
            x = jnp.concatenate([x, _masked_avg(x, mask)], 2)
            x = _graph_conv1x1(x, blk["fc1"], True)
            v = x + x_in
    return _final_conv(f, params["conv2"])


# speedup vs baseline: 3.7141x; 1.0641x over previous
import functools

import jax
import jax.numpy as jnp
from jax import lax
from jax.experimental import pallas as pl
from jax.experimental.pallas import tpu as pltpu
from jax.experimental.pallas import tpu_sc as plsc
from jax._src import core as _jax_core
from jax._src.pallas import core as _pallas_core


def _to_device_space(x):
    return _pallas_core.with_memory_space_constraint_p.bind(
        x, memory_space=_jax_core.MemorySpace.Device)

B = 1
N = 10000
NF = 20000
C = 128
NNZ = 960000

_CHUNK = 128
_NNZ_P = 983040
_NCHUNK = _NNZ_P // _CHUNK
_PER_SUB = _NCHUNK // 16
_SUP = 24
_NSUP = _PER_SUB // _SUP


def _pad_edges(a):
    return jnp.concatenate([a, jnp.zeros((_NNZ_P - NNZ,), a.dtype)])


def _make_sc_spmm(M, R, npass):
    RH = R // npass
    stripe = -(-(RH // 16 + 1) // 8) * 8
    RHp = 16 * stripe

    grid = plsc.VectorSubcoreMesh(core_axis_name="c", subcore_axis_name="s")

    GB = _CHUNK * 16 * 4
    STB = 3 * _SUP * _CHUNK * 4

    @functools.partial(
        pl.kernel,
        out_type=pltpu.HBM((2, npass, RHp, 16), jnp.float32),
        mesh=grid,
        compiler_params=pltpu.CompilerParams(use_tc_tiling_on_sc=False),
        scratch_types=[
            pltpu.VMEM_SHARED((RHp, 16), jnp.float32),
            pltpu.VMEM((stripe, 16), jnp.float32),
            pltpu.VMEM((_SUP, _CHUNK), jnp.int32),
            pltpu.VMEM((_SUP, _CHUNK), jnp.float32),
            pltpu.VMEM((_SUP, _CHUNK), jnp.int32),
            pltpu.VMEM((_SUP, _CHUNK), jnp.int32),
            pltpu.VMEM((_SUP, _CHUNK), jnp.float32),
            pltpu.VMEM((_SUP, _CHUNK), jnp.int32),
            pltpu.VMEM((_CHUNK, 16), jnp.float32),
            pltpu.VMEM((_CHUNK, 16), jnp.float32),
            pltpu.SemaphoreType.DMA,
            pltpu.SemaphoreType.DMA,
            pltpu.SemaphoreType.DMA,
            pltpu.SemaphoreType.DMA,
            pltpu.SemaphoreType.DMA,
            pltpu.SemaphoreType.DMA,
        ],
    )
    def spmm(tbl, cols, vals, rows, out, acc, zbuf,
             ci0, vb0, ri0, ci1, vb1, ri1, g0, g1,
             gs0, gs1, ss0, ss1, st0, st1):
        c = lax.axis_index("c")
        s = lax.axis_index("s")
        z16 = jnp.zeros((16,), jnp.float32)
        sets = ((ci0, vb0, ri0, st0), (ci1, vb1, ri1, st1))
        gbuf = (g0, g1)
        gsem = (gs0, gs1)
        ssem = (ss0, ss1)

        def zrow(r, carry):
            zbuf[r, pl.ds(0, 16)] = z16
            return carry

        lax.fori_loop(0, stripe, zrow, 0)

        cM = (c * M).astype(jnp.int32)
        base_chunk = s * _PER_SUB

        def stage_start(t, si):
            ci, vb, ri, sem = sets[si]
            row0 = base_chunk + t * _SUP
            pltpu.async_copy(cols.at[pl.ds(row0, _SUP)], ci, sem)
            pltpu.async_copy(vals.at[pl.ds(row0, _SUP)], vb, sem)
            pltpu.async_copy(rows.at[pl.ds(row0, _SUP)], ri, sem)

        def stage_wait(si):
            ci, vb, ri, sem = sets[si]
            for dst, src in ((ci, cols), (vb, vals), (ri, rows)):
                pltpu.make_async_copy(src.at[pl.ds(0, _SUP)], dst, sem).wait()

        def chunk_wait(sem, b):
            pltpu.make_async_copy(tbl.at[pl.ds(0, _CHUNK)], gbuf[b], sem).wait()

        def transform(si, q, lo):
            ci, _, ri, _ = sets[si]
            for k in range(_CHUNK // 16):
                sl = pl.ds(k * 16, 16)
                ci[q, sl] = ci[q, sl] + cM
                if npass > 1:
                    lr = ri[q, sl] - lo
                    ok = (lr >= 0) & (lr < RH)
                    ri[q, sl] = jnp.where(ok, lr, jnp.int32(RH))

        def gather_start(si, q, b):
            pltpu.async_copy(tbl.at[sets[si][0].at[q]], gbuf[b], gsem[b])

        def process(si, q, b):
            _, vb, ri, _ = sets[si]
            g = gbuf[b]
            for k in range(_CHUNK // 16):
                val16 = vb[q, pl.ds(k * 16, 16)]
                for j in range(16):
                    e = k * 16 + j
                    g[e, pl.ds(0, 16)] = g[e, pl.ds(0, 16)] * val16[j]
            pltpu.async_copy(g, acc.at[ri.at[q]], ssem[b], add=True)

        for p in range(npass):
            lo = jnp.int32(p * RH)
            pltpu.sync_copy(zbuf, acc.at[pl.ds(s * stripe, stripe)])
            plsc.subcore_barrier()
            pltpu.sync_copy(cols.at[pl.ds(base_chunk, _SUP)], ci0)
            pltpu.sync_copy(vals.at[pl.ds(base_chunk, _SUP)], vb0)
            pltpu.sync_copy(rows.at[pl.ds(base_chunk, _SUP)], ri0)

            def superchunk(si):
                def run(t):
                    @pl.when(t > 0)
                    def _():
                        chunk_wait(ssem[0], 0)

                    transform(si, 0, lo)
                    gather_start(si, 0, 0)

                    def body2(i2, carry):
                        for qoff, b in ((0, 0), (1, 1)):
                            q = 2 * i2 + qoff
                            o = 1 - b

                            @pl.when((q < _SUP - 1) & ((t > 0) | (q > 0)))
                            def _():
                                chunk_wait(ssem[o], o)

                            @pl.when(q < _SUP - 1)
                            def _():
                                transform(si, q + 1, lo)
                                gather_start(si, q + 1, o)

                            chunk_wait(gsem[b], b)
                            process(si, q, b)
                        return carry

                    lax.fori_loop(0, _SUP // 2, body2, 0)
                return run

            run0 = superchunk(0)
            run1 = superchunk(1)

            def pair(tt, carry):
                t0 = 2 * tt

                @pl.when(tt > 0)
                def _():
                    stage_wait(0)

                stage_start(t0 + 1, 1)
                run0(t0)
                stage_wait(1)

                @pl.when(tt < _NSUP // 2 - 1)
                def _():
                    stage_start(t0 + 2, 0)

                run1(t0 + 1)
                return carry

            lax.fori_loop(0, _NSUP // 2, pair, 0)
            chunk_wait(ss0, 0)
            chunk_wait(ss1, 1)
            plsc.subcore_barrier()
            pltpu.sync_copy(acc.at[pl.ds(s * stripe, stripe)],
                            out.at[c, p, pl.ds(s * stripe, stripe)])
            if p + 1 < npass:
                plsc.subcore_barrier()

    return spmm


_sc_spmm_di = _make_sc_spmm(4 * N, 4 * NF, 2)
_sc_spmm_dia = _make_sc_spmm(4 * NF, 4 * N, 1)


def _sc_sparse_bmm(spmm, x4, cols2, vals2, rows2, R, npass):
    tbl = jnp.concatenate([x4[:, :16], x4[:, 16:]], 0)
    out2 = _to_device_space(spmm(tbl, cols2, vals2, rows2))
    RH = R // npass
    halves = [out2[:, p, :RH] for p in range(npass)]
    full = jnp.concatenate(halves, 1)
    return jnp.concatenate([full[0], full[1]], 1)


def _elu(x):
    return jnp.where(x > 0, x, jnp.exp(jnp.minimum(x, 0.0)) - 1.0)


def _bn(x, gamma, beta):
    mu = jnp.mean(x, 0, keepdims=True)
    var = jnp.mean(jnp.square(x - mu), 0, keepdims=True)
    return (x - mu) * jax.lax.rsqrt(var + 1e-5) * gamma + beta


def _make_convbn2(Nr, elu_a, with_b, avg_b, with_res):
    def body(*refs):
        i = 0
        a_ref = refs[i]; i += 1
        b_ref = mask_ref = res_ref = None
        if avg_b:
            mask_ref = refs[i]; i += 1
        elif with_b:
            b_ref = refs[i]; i += 1
        wa_ref = refs[i]; i += 1
        wb_ref = refs[i]; i += 1
        bias_ref = refs[i]; i += 1
        ga_ref = refs[i]; i += 1
        bta_ref = refs[i]; i += 1
        gb_ref = refs[i]; i += 1
        btb_ref = refs[i]; i += 1
        if with_res:
            res_ref = refs[i]; i += 1
        o_ref = refs[i]

        A = a_ref[...]
        if elu_a:
            A = _elu(A)
        if avg_b:
            m = mask_ref[...]
            avg = jnp.sum(A * m, 0, keepdims=True) / jnp.sum(m)
            Bm = jnp.broadcast_to(avg, (Nr, C))
        elif with_b:
            Bm = b_ref[...]
        Y = jnp.dot(_bn(A, ga_ref[...], bta_ref[...]), wa_ref[...],
                    preferred_element_type=jnp.float32)
        if avg_b or with_b:
            Y = Y + jnp.dot(_bn(Bm, gb_ref[...], btb_ref[...]), wb_ref[...],
                            preferred_element_type=jnp.float32)
        Y = Y + bias_ref[...]
        if with_res:
            Y = Y + res_ref[...]
        o_ref[...] = Y

    def call(a, second, p, res):
        W = p["W"]
        gamma = p["gamma"]
        beta = p["beta"]
        args = [a]
        if avg_b or with_b:
            args.append(second)
        args += [W[:C], W[C:] if W.shape[0] == 2 * C else jnp.zeros((C, C), jnp.float32),
                 p["b"].reshape(1, C),
                 gamma[:C].reshape(1, C), beta[:C].reshape(1, C),
                 (gamma[C:] if gamma.shape[0] == 2 * C else jnp.zeros((C,))).reshape(1, C),
                 (beta[C:] if beta.shape[0] == 2 * C else jnp.zeros((C,))).reshape(1, C)]
        if with_res:
            args.append(res)
        return pl.pallas_call(
            body,
            out_shape=jax.ShapeDtypeStruct((Nr, C), jnp.float32),
        )(*args)

    return call


_conv_even_v = _make_convbn2(N, True, True, False, True)
_conv_even_f = _make_convbn2(NF, True, True, False, False)
_conv_odd0 = _make_convbn2(N, True, False, True, False)
_conv_odd1 = _make_convbn2(N, True, False, True, True)


def _conv1_body(x_ref, w_ref, b_ref, o_ref):
    x = x_ref[...]
    w = w_ref[...]
    acc = b_ref[...]
    for k in range(3):
        acc = acc + x[:, k:k + 1] * w[k:k + 1, :]
    o_ref[...] = acc


def _conv1(x, p):
    return pl.pallas_call(
        _conv1_body,
        out_shape=jax.ShapeDtypeStruct((N, C), jnp.float32),
    )(x, p["W"], p["b"].reshape(1, C))


def _final_conv_body(x_ref, w_ref, b_ref, g_ref, bt_ref, o_ref):
    x = _elu(x_ref[...])
    mu = jnp.mean(x, axis=0, keepdims=True)
    var = jnp.mean(jnp.square(x - mu), axis=0, keepdims=True)
    xn = (x - mu) * jax.lax.rsqrt(var + 1e-5)
    xn = xn * g_ref[...] + bt_ref[...]
    o = xn @ w_ref[...] + b_ref[0, 0]
    o_ref[...] = jnp.broadcast_to(o, (NF, 128))


def _final_conv(f, p):
    x = f
    w = p["W"].reshape(C, 1)
    out = pl.pallas_call(
        _final_conv_body,
        out_shape=jax.ShapeDtypeStruct((NF, 128), jnp.float32),
    )(x, w, p["b"].reshape(1, 1), p["gamma"].reshape(1, C), p["beta"].reshape(1, C))
    return out[:, :1].reshape(B, NF, 1)


def kernel(inputs, mask, Di_vals, DiA_vals, params, Di_rows, Di_cols, DiA_rows, DiA_cols):
    di_c = _pad_edges(Di_cols.astype(jnp.int32)).reshape(_NCHUNK, _CHUNK)
    di_v = _pad_edges(Di_vals).reshape(_NCHUNK, _CHUNK)
    di_r = _pad_edges(Di_rows.astype(jnp.int32)).reshape(_NCHUNK, _CHUNK)
    da_c = _pad_edges(DiA_cols.astype(jnp.int32)).reshape(_NCHUNK, _CHUNK)
    da_v = _pad_edges(DiA_vals).reshape(_NCHUNK, _CHUNK)
    da_r = _pad_edges(DiA_rows.astype(jnp.int32)).reshape(_NCHUNK, _CHUNK)
    mask2 = mask.reshape(N, 1)
    v = _conv1(inputs.reshape(N, 3), params["conv1"])
    f = jnp.zeros((NF, C), jnp.float32)
    for i in range(16):
        blk = params["blocks"][i]
        if i % 2 == 0:
            x4 = _elu(v).reshape(4 * N, C // 4)
            f4 = _elu(f).reshape(4 * NF, C // 4)
            Dv = _sc_sparse_bmm(_sc_spmm_di, x4, di_c, di_v, di_r, 4 * NF, 2).reshape(NF, C)
            DAf = _sc_sparse_bmm(_sc_spmm_dia, f4, da_c, da_v, da_r, 4 * N, 1).reshape(N, C)
            v = _conv_even_v(v, DAf, blk["fc0"], v)
            f = _conv_even_f(f, Dv, blk["fc1"], None)
        else:
            t = _conv_odd0(v, mask2, blk["fc0"], None)
            v = _conv_odd1(t, mask2, blk["fc1"], v)
    return _final_conv(f, params["conv2"])
